# baseline (speedup 1.0000x reference)
import functools

import jax
import jax.numpy as jnp
from jax import lax
from jax.experimental import pallas as pl
from jax.experimental.pallas import tpu as pltpu
from jax.experimental.pallas import tpu_sc as plsc

N = 10000
E = 160000
D = 256

NC = 2
NS = 16
L = 16
NW = NC * NS

EPW = E // NW
CG = 40
NCHUNK_G = EPW // CG

OWN = 320
NPAD = NW * OWN
CSC = 2000
NCHUNK_S = E // CSC
GRP = 128
EPAD = E + GRP


def _mish(x):
    return x * jnp.tanh(jax.nn.softplus(x))


def _ln(x, g, b):
    mu = jnp.mean(x, axis=-1, keepdims=True)
    var = jnp.var(x, axis=-1, keepdims=True)
    return (x - mu) / jnp.sqrt(var + 1e-5) * g + b


def _node_body(c_ref, wq, bq, gq, nq, wk, bk, gk, nk,
               pw1, pb1, pg, pe, pw2, pb2,
               bw1, bb1, bg, be, bw2, bb2,
               cw1, cb1, cw2,
               q_out, kpb_out, x_out):
    c = c_ref[:]
    q = _mish(_ln(jnp.dot(c, wq[:], preferred_element_type=jnp.float32) + bq[:],
                  gq[:], nq[:]))
    q_out[:] = q
    k = _mish(_ln(jnp.dot(c, wk[:], preferred_element_type=jnp.float32) + bk[:],
                  gk[:], nk[:]))
    pem = jnp.dot(
        _mish(_ln(jnp.dot(c, pw1[:], preferred_element_type=jnp.float32) + pb1[:],
                  pg[:], pe[:])),
        pw2[:], preferred_element_type=jnp.float32) + pb2[:]
    peb = jnp.dot(
        _mish(_ln(jnp.dot(c, bw1[:], preferred_element_type=jnp.float32) + bb1[:],
                  bg[:], be[:])),
        bw2[:], preferred_element_type=jnp.float32) + bb2[:]
    kpb_out[:, 0:D] = k
    kpb_out[:, D:2 * D] = pem
    kpb_out[:, 2 * D:3 * D] = peb
    x_out[:] = jnp.dot(_mish(jnp.dot(c, cw1[:], preferred_element_type=jnp.float32)
                             + cb1[:]),
                       cw2[:], preferred_element_type=jnp.float32)


def _node_call(coord, p):
    blk = 512
    grid = (pl.cdiv(N, blk),)
    row_spec = pl.BlockSpec((blk, D), lambda i: (i, 0))
    w_spec = pl.BlockSpec((D, D), lambda i: (0, 0))
    v_spec = pl.BlockSpec((1, D), lambda i: (0, 0))
    vecs = lambda *names: [p[n].reshape(1, D) for n in names]
    return pl.pallas_call(
        _node_body,
        grid=grid,
        in_specs=[row_spec,
                  w_spec, v_spec, v_spec, v_spec,
                  w_spec, v_spec, v_spec, v_spec,
                  w_spec, v_spec, v_spec, v_spec, w_spec, v_spec,
                  w_spec, v_spec, v_spec, v_spec, w_spec, v_spec,
                  w_spec, v_spec, w_spec],
        out_specs=[row_spec,
                   pl.BlockSpec((blk, 3 * D), lambda i: (i, 0)),
                   row_spec],
        out_shape=[jax.ShapeDtypeStruct((N, D), jnp.float32),
                   jax.ShapeDtypeStruct((N, 3 * D), jnp.float32),
                   jax.ShapeDtypeStruct((N, D), jnp.float32)],
    )(coord,
      p['Wq'], *vecs('bq', 'gq', 'betaq'),
      p['Wk'], *vecs('bk', 'gk', 'betak'),
      p['pm_W1'], *vecs('pm_b1', 'pm_g', 'pm_be'), p['pm_W2'], p['pm_b2'].reshape(1, D),
      p['pb_W1'], *vecs('pb_b1', 'pb_g', 'pb_be'), p['pb_W2'], p['pb_b2'].reshape(1, D),
      p['cm_W1'], p['cm_b1'].reshape(1, D), p['cm_W2'])


def _edge_rel_compute(qrows_v, kpbrows_v, out_v, b):
    @functools.partial(plsc.parallel_loop, b * CG, (b + 1) * CG, unroll=2)
    def edge_body(e):
        dvs = []
        acc = jnp.zeros((L,), jnp.float32)
        for j in range(D // L):
            qv = qrows_v[e, pl.ds(j * L, L)]
            kv = kpbrows_v[e, pl.ds(j * L, L)]
            dv = qv - kv
            dvs.append(dv)
            acc = acc + dv * dv
        iota = lax.iota(jnp.int32, L)
        for sh in (8, 4, 2, 1):
            acc = acc + acc.at[iota ^ sh].get(mode='promise_in_bounds',
                                              unique_indices=True)
        x = acc + 1e-8
        yi = jnp.int32(0x5F3759DF) - (plsc.bitcast(x, jnp.int32) >> 1)
        y = plsc.bitcast(yi, jnp.float32)
        for _ in range(3):
            y = y * (1.5 - 0.5 * x * y * y)
        for j in range(D // L):
            pv = kpbrows_v[e, pl.ds(D + j * L, L)]
            bv = kpbrows_v[e, pl.ds(2 * D + j * L, L)]
            out_v[e, pl.ds(j * L, L)] = dvs[j] * y * pv + bv


def _gather_body(q_hbm, kpb_hbm, src_hbm, dst_hbm, rel_hbm,
                 sidx_v, didx_v, qrows_v, kpbrows_v, out_v,
                 semg0, semg1, semw0, semw1):
    c = lax.axis_index("c")
    s = lax.axis_index("s")
    wid = s * NC + c
    base_w = wid * EPW
    semg = (semg0, semg1)
    semw = (semw0, semw1)

    pltpu.sync_copy(src_hbm.at[pl.ds(base_w, EPW)], sidx_v)
    pltpu.sync_copy(dst_hbm.at[pl.ds(base_w, EPW)], didx_v)

    def start_gather(ci, b):
        pltpu.async_copy(q_hbm.at[sidx_v.at[pl.ds(ci * CG, CG)]],
                         qrows_v.at[pl.ds(b * CG, CG)], semg[b])
        pltpu.async_copy(kpb_hbm.at[didx_v.at[pl.ds(ci * CG, CG)]],
                         kpbrows_v.at[pl.ds(b * CG, CG)], semg[b])

    def wait_gather(ci, b):
        pltpu.make_async_copy(q_hbm.at[sidx_v.at[pl.ds(ci * CG, CG)]],
                              qrows_v.at[pl.ds(b * CG, CG)], semg[b]).wait()
        pltpu.make_async_copy(kpb_hbm.at[didx_v.at[pl.ds(ci * CG, CG)]],
                              kpbrows_v.at[pl.ds(b * CG, CG)], semg[b]).wait()

    def start_write(ci, b):
        pltpu.async_copy(out_v.at[pl.ds(b * CG, CG)],
                         rel_hbm.at[pl.ds(base_w + ci * CG, CG)], semw[b])

    def wait_write(ci, b):
        pltpu.make_async_copy(out_v.at[pl.ds(b * CG, CG)],
                              rel_hbm.at[pl.ds(base_w + ci * CG, CG)],
                              semw[b]).wait()

    start_gather(0, 0)
    start_gather(1, 1)
    start_write(0, 0)
    start_write(1, 1)

    def outer(cj, carry):
        for b in range(2):
            ci = cj * 2 + b
            wait_gather(ci, b)
            wait_write(ci, b)
            _edge_rel_compute(qrows_v, kpbrows_v, out_v, b)
            start_write(ci, b)

            @pl.when(ci + 2 < NCHUNK_G)
            def _():
                start_gather(ci + 2, b)

        return carry

    lax.fori_loop(0, NCHUNK_G // 2, outer, 0)

    ci = NCHUNK_G - 1
    wait_gather(ci, 0)
    wait_write(ci, 0)
    _edge_rel_compute(qrows_v, kpbrows_v, out_v, 0)
    start_write(ci, 0)
    wait_write(ci, 0)
    wait_write(ci, 1)


def _gather_call(q, kpb, src, dst):
    mesh = plsc.VectorSubcoreMesh(core_axis_name="c", subcore_axis_name="s",
                                  num_cores=NC, num_subcores=NS)
    return pl.kernel(
        _gather_body,
        out_type=jax.ShapeDtypeStruct((E, D), jnp.float32),
        mesh=mesh,
        compiler_params=pltpu.CompilerParams(needs_layout_passes=False),
        scratch_types=[
            pltpu.VMEM((EPW,), jnp.int32),
            pltpu.VMEM((EPW,), jnp.int32),
            pltpu.VMEM((2 * CG, D), jnp.float32),
            pltpu.VMEM((2 * CG, 3 * D), jnp.float32),
            pltpu.VMEM((2 * CG, D), jnp.float32),
            pltpu.SemaphoreType.DMA,
            pltpu.SemaphoreType.DMA,
            pltpu.SemaphoreType.DMA,
            pltpu.SemaphoreType.DMA,
        ],
    )(q, kpb, src, dst)


def _edge_body(rel_ref, w1, b1, g, be, w2, b2, ew_out):
    blk = ew_out.shape[0]
    t = _mish(_ln(jnp.dot(rel_ref[:], w1[:], preferred_element_type=jnp.float32)
                  + b1[:], g[:], be[:]))
    w = jnp.dot(t, w2[:], preferred_element_type=jnp.float32) + b2[:]
    row = (pl.program_id(0) * blk
           + jax.lax.broadcasted_iota(jnp.int32, (blk, 1), 0))
    ew_out[:] = jnp.where(row < E, jnp.exp(w), 0.0)


def _edge_call(rel, p):
    blk = 512
    grid = (pl.cdiv(EPAD, blk),)
    row_spec = pl.BlockSpec((blk, D), lambda i: (i, 0))
    w_spec = pl.BlockSpec((D, D), lambda i: (0, 0))
    v_spec = pl.BlockSpec((1, D), lambda i: (0, 0))
    return pl.pallas_call(
        _edge_body,
        grid=grid,
        in_specs=[row_spec, w_spec, v_spec, v_spec, v_spec, w_spec, v_spec],
        out_specs=row_spec,
        out_shape=jax.ShapeDtypeStruct((EPAD, D), jnp.float32),
    )(rel, p['we_W1'], p['we_b1'].reshape(1, D), p['we_g'].reshape(1, D),
      p['we_be'].reshape(1, D), p['we_W2'], p['we_b2'].reshape(1, D))


def _scatter_body(ew_hbm, dst_hbm, s_out_hbm,
                  didx_v, eid_v, nloc_v, eid_g, rows_v, acc_v, sem):
    c = lax.axis_index("c")
    s = lax.axis_index("s")
    wid = s * NC + c
    lo = wid * OWN

    def zbody(i, carry):
        for j in range(D // L):
            acc_v[i, pl.ds(j * L, L)] = jnp.zeros((L,), jnp.float32)
        return carry

    lax.fori_loop(0, OWN, zbody, 0)

    iota = lax.iota(jnp.int32, L)

    def chunk_body(ci, carry):
        base = ci * CSC
        pltpu.sync_copy(dst_hbm.at[pl.ds(base, CSC)], didx_v)

        def scan_body(i, m):
            dv = didx_v[pl.ds(i * L, L)]
            lv = dv - lo
            ok = (lv >= 0) & (lv < OWN)
            plsc.store_compressed(eid_v.at[pl.ds(m, L)], base + i * L + iota,
                                  mask=ok)
            plsc.store_compressed(nloc_v.at[pl.ds(m, L)], lv, mask=ok)
            pc = plsc.all_reduce_population_count(ok)
            return m + pc[0]

        m = lax.fori_loop(0, CSC // L, scan_body, jnp.int32(0))

        for t in range(GRP // L):
            eid_v[pl.ds(m + t * L, L)] = jnp.full((L,), E, jnp.int32)
            nloc_v[pl.ds(m + t * L, L)] = jnp.zeros((L,), jnp.int32)

        def group_work(g):
            for r in range(GRP // L):
                eid_g[pl.ds(r * L, L)] = eid_v[pl.ds(g * GRP + r * L, L)]
            pltpu.async_copy(ew_hbm.at[eid_g], rows_v, sem).wait()

            def blk16(k, carry3):
                nv = nloc_v[pl.ds(g * GRP + k * L, L)]
                for r in range(L):
                    n = nv[r]
                    row = k * L + r
                    for j in range(D // L):
                        plsc.addupdate(acc_v.at[n, pl.ds(j * L, L)],
                                       rows_v[row, pl.ds(j * L, L)])
                return carry3

            lax.fori_loop(0, GRP // L, blk16, 0)

        ng = (m + GRP - 1) // GRP
        for gs in range(2):
            @pl.when(gs < ng)
            def _():
                group_work(gs)

        def group_body(g, carry2):
            group_work(g)
            return carry2

        lax.fori_loop(2, ng, group_body, 0)
        return carry

    lax.fori_loop(0, NCHUNK_S, chunk_body, 0)
    pltpu.sync_copy(acc_v, s_out_hbm.at[pl.ds(lo, OWN)])


def _scatter_call(ew, dst):
    mesh = plsc.VectorSubcoreMesh(core_axis_name="c", subcore_axis_name="s",
                                  num_cores=NC, num_subcores=NS)
    out = pl.kernel(
        _scatter_body,
        out_type=jax.ShapeDtypeStruct((NPAD, D), jnp.float32),
        mesh=mesh,
        compiler_params=pltpu.CompilerParams(needs_layout_passes=False),
        scratch_types=[
            pltpu.VMEM((CSC,), jnp.int32),
            pltpu.VMEM((CSC + 2 * GRP,), jnp.int32),
            pltpu.VMEM((CSC + 2 * GRP,), jnp.int32),
            pltpu.VMEM((GRP,), jnp.int32),
            pltpu.VMEM((GRP, D), jnp.float32),
            pltpu.VMEM((OWN, D), jnp.float32),
            pltpu.SemaphoreType.DMA,
        ],
    )(ew, dst)
    return out[:N]


def _colsum_body(s_ref, out_ref):
    out_ref[:] = jnp.sum(s_ref[:], axis=0, keepdims=True)


def _colsum_call(S):
    return pl.pallas_call(
        _colsum_body,
        out_shape=jax.ShapeDtypeStruct((1, D), jnp.float32),
    )(S)


def _final_body(nf_ref, s_ref, cs_ref, w1a, w1b, b1, w2, b2, h_out):
    hn = s_ref[:] / cs_ref[:]
    u = (jnp.dot(nf_ref[:], w1a[:], preferred_element_type=jnp.float32)
         + jnp.dot(hn, w1b[:], preferred_element_type=jnp.float32) + b1[:])
    h_out[:] = jnp.dot(_mish(u), w2[:], preferred_element_type=jnp.float32) + b2[:]


def _final_call(node_feat, S, colsum, p):
    blk = 512
    grid = (pl.cdiv(N, blk),)
    row_spec = pl.BlockSpec((blk, D), lambda i: (i, 0))
    w_spec = pl.BlockSpec((D, D), lambda i: (0, 0))
    v_spec = pl.BlockSpec((1, D), lambda i: (0, 0))
    return pl.pallas_call(
        _final_body,
        grid=grid,
        in_specs=[row_spec, row_spec, v_spec,
                  w_spec, w_spec, v_spec, w_spec, v_spec],
        out_specs=row_spec,
        out_shape=jax.ShapeDtypeStruct((N, D), jnp.float32),
    )(node_feat, S, colsum,
      p['nm_W1'][:D], p['nm_W1'][D:], p['nm_b1'].reshape(1, D),
      p['nm_W2'], p['nm_b2'].reshape(1, D))


@jax.jit
def _run(node_feat, coord_feat, params, edge_index):
    src = edge_index[0]
    dst = edge_index[1]
    q, kpb, x = _node_call(coord_feat, params)
    rel = _gather_call(q, kpb, src, dst)
    ew = _edge_call(rel, params)
    S = _scatter_call(ew, dst)
    colsum = _colsum_call(S)
    h = _final_call(node_feat, S, colsum, params)
    return (h, x)


def kernel(node_feat, coord_feat, params, edge_index):
    return _run(node_feat, coord_feat, params, edge_index)


# speedup vs baseline: 1.9058x; 1.9058x over previous
import functools

import jax
import jax.numpy as jnp
from jax import lax
from jax.experimental import pallas as pl
from jax.experimental.pallas import tpu as pltpu
from jax.experimental.pallas import tpu_sc as plsc

N = 10000
E = 160000
D = 256

NC = 2
NS = 16
L = 16
NW = NC * NS

EPW = E // NW
CG = 40
NCHUNK_G = EPW // CG

OWN = 320
NPAD = NW * OWN
CSC = 2000
NCHUNK_S = E // CSC
GRP = 64
EPAD = E + GRP


def _mish(x):
    return x * jnp.tanh(jax.nn.softplus(x))


def _ln(x, g, b):
    mu = jnp.mean(x, axis=-1, keepdims=True)
    var = jnp.var(x, axis=-1, keepdims=True)
    return (x - mu) / jnp.sqrt(var + 1e-5) * g + b


def _node_body(c_ref, wq, bq, gq, nq, wk, bk, gk, nk,
               pw1, pb1, pg, pe, pw2, pb2,
               bw1, bb1, bg, be, bw2, bb2,
               cw1, cb1, cw2,
               q_out, kpb_out, x_out):
    c = c_ref[:]
    q = _mish(_ln(jnp.dot(c, wq[:], preferred_element_type=jnp.float32) + bq[:],
                  gq[:], nq[:]))
    q_out[:] = q
    k = _mish(_ln(jnp.dot(c, wk[:], preferred_element_type=jnp.float32) + bk[:],
                  gk[:], nk[:]))
    pem = jnp.dot(
        _mish(_ln(jnp.dot(c, pw1[:], preferred_element_type=jnp.float32) + pb1[:],
                  pg[:], pe[:])),
        pw2[:], preferred_element_type=jnp.float32) + pb2[:]
    peb = jnp.dot(
        _mish(_ln(jnp.dot(c, bw1[:], preferred_element_type=jnp.float32) + bb1[:],
                  bg[:], be[:])),
        bw2[:], preferred_element_type=jnp.float32) + bb2[:]
    kpb_out[:, 0:D] = k
    kpb_out[:, D:2 * D] = pem
    kpb_out[:, 2 * D:3 * D] = peb
    x_out[:] = jnp.dot(_mish(jnp.dot(c, cw1[:], preferred_element_type=jnp.float32)
                             + cb1[:]),
                       cw2[:], preferred_element_type=jnp.float32)


def _node_call(coord, p):
    blk = 512
    grid = (pl.cdiv(N, blk),)
    row_spec = pl.BlockSpec((blk, D), lambda i: (i, 0))
    w_spec = pl.BlockSpec((D, D), lambda i: (0, 0))
    v_spec = pl.BlockSpec((1, D), lambda i: (0, 0))
    vecs = lambda *names: [p[n].reshape(1, D) for n in names]
    return pl.pallas_call(
        _node_body,
        grid=grid,
        in_specs=[row_spec,
                  w_spec, v_spec, v_spec, v_spec,
                  w_spec, v_spec, v_spec, v_spec,
                  w_spec, v_spec, v_spec, v_spec, w_spec, v_spec,
                  w_spec, v_spec, v_spec, v_spec, w_spec, v_spec,
                  w_spec, v_spec, w_spec],
        out_specs=[row_spec,
                   pl.BlockSpec((blk, 3 * D), lambda i: (i, 0)),
                   row_spec],
        out_shape=[jax.ShapeDtypeStruct((N, D), jnp.float32),
                   jax.ShapeDtypeStruct((N, 3 * D), jnp.float32),
                   jax.ShapeDtypeStruct((N, D), jnp.float32)],
    )(coord,
      p['Wq'], *vecs('bq', 'gq', 'betaq'),
      p['Wk'], *vecs('bk', 'gk', 'betak'),
      p['pm_W1'], *vecs('pm_b1', 'pm_g', 'pm_be'), p['pm_W2'], p['pm_b2'].reshape(1, D),
      p['pb_W1'], *vecs('pb_b1', 'pb_g', 'pb_be'), p['pb_W2'], p['pb_b2'].reshape(1, D),
      p['cm_W1'], p['cm_b1'].reshape(1, D), p['cm_W2'])


def _edge_rel_compute(qrows_v, kpbrows_v, out_v, b):
    @functools.partial(plsc.parallel_loop, b * CG, (b + 1) * CG, unroll=2)
    def edge_body(e):
        dvs = []
        acc = jnp.zeros((L,), jnp.float32)
        for j in range(D // L):
            qv = qrows_v[e, pl.ds(j * L, L)]
            kv = kpbrows_v[e, pl.ds(j * L, L)]
            dv = qv - kv
            dvs.append(dv)
            acc = acc + dv * dv
        iota = lax.iota(jnp.int32, L)
        for sh in (8, 4, 2, 1):
            acc = acc + acc.at[iota ^ sh].get(mode='promise_in_bounds',
                                              unique_indices=True)
        x = acc + 1e-8
        yi = jnp.int32(0x5F3759DF) - (plsc.bitcast(x, jnp.int32) >> 1)
        y = plsc.bitcast(yi, jnp.float32)
        for _ in range(3):
            y = y * (1.5 - 0.5 * x * y * y)
        for j in range(D // L):
            pv = kpbrows_v[e, pl.ds(D + j * L, L)]
            bv = kpbrows_v[e, pl.ds(2 * D + j * L, L)]
            out_v[e, pl.ds(j * L, L)] = dvs[j] * y * pv + bv


def _gather_body(q_hbm, kpb_hbm, src_hbm, dst_hbm, rel_hbm,
                 sidx_v, didx_v, qrows_v, kpbrows_v, out_v,
                 semg0, semg1, semw0, semw1):
    c = lax.axis_index("c")
    s = lax.axis_index("s")
    wid = s * NC + c
    base_w = wid * EPW
    semg = (semg0, semg1)
    semw = (semw0, semw1)

    pltpu.sync_copy(src_hbm.at[pl.ds(base_w, EPW)], sidx_v)
    pltpu.sync_copy(dst_hbm.at[pl.ds(base_w, EPW)], didx_v)

    def start_gather(ci, b):
        pltpu.async_copy(q_hbm.at[sidx_v.at[pl.ds(ci * CG, CG)]],
                         qrows_v.at[pl.ds(b * CG, CG)], semg[b])
        pltpu.async_copy(kpb_hbm.at[didx_v.at[pl.ds(ci * CG, CG)]],
                         kpbrows_v.at[pl.ds(b * CG, CG)], semg[b])

    def wait_gather(ci, b):
        pltpu.make_async_copy(q_hbm.at[sidx_v.at[pl.ds(ci * CG, CG)]],
                              qrows_v.at[pl.ds(b * CG, CG)], semg[b]).wait()
        pltpu.make_async_copy(kpb_hbm.at[didx_v.at[pl.ds(ci * CG, CG)]],
                              kpbrows_v.at[pl.ds(b * CG, CG)], semg[b]).wait()

    def start_write(ci, b):
        pltpu.async_copy(out_v.at[pl.ds(b * CG, CG)],
                         rel_hbm.at[pl.ds(base_w + ci * CG, CG)], semw[b])

    def wait_write(ci, b):
        pltpu.make_async_copy(out_v.at[pl.ds(b * CG, CG)],
                              rel_hbm.at[pl.ds(base_w + ci * CG, CG)],
                              semw[b]).wait()

    start_gather(0, 0)
    start_gather(1, 1)
    start_write(0, 0)
    start_write(1, 1)

    def outer(cj, carry):
        for b in range(2):
            ci = cj * 2 + b
            wait_gather(ci, b)
            wait_write(ci, b)
            _edge_rel_compute(qrows_v, kpbrows_v, out_v, b)
            start_write(ci, b)

            @pl.when(ci + 2 < NCHUNK_G)
            def _():
                start_gather(ci + 2, b)

        return carry

    lax.fori_loop(0, NCHUNK_G // 2, outer, 0)

    ci = NCHUNK_G - 1
    wait_gather(ci, 0)
    wait_write(ci, 0)
    _edge_rel_compute(qrows_v, kpbrows_v, out_v, 0)
    start_write(ci, 0)
    wait_write(ci, 0)
    wait_write(ci, 1)


def _gather_call(q, kpb, src, dst):
    mesh = plsc.VectorSubcoreMesh(core_axis_name="c", subcore_axis_name="s",
                                  num_cores=NC, num_subcores=NS)
    return pl.kernel(
        _gather_body,
        out_type=jax.ShapeDtypeStruct((E, D), jnp.float32),
        mesh=mesh,
        compiler_params=pltpu.CompilerParams(needs_layout_passes=False),
        scratch_types=[
            pltpu.VMEM((EPW,), jnp.int32),
            pltpu.VMEM((EPW,), jnp.int32),
            pltpu.VMEM((2 * CG, D), jnp.float32),
            pltpu.VMEM((2 * CG, 3 * D), jnp.float32),
            pltpu.VMEM((2 * CG, D), jnp.float32),
            pltpu.SemaphoreType.DMA,
            pltpu.SemaphoreType.DMA,
            pltpu.SemaphoreType.DMA,
            pltpu.SemaphoreType.DMA,
        ],
    )(q, kpb, src, dst)


def _edge_body(rel_ref, w1, b1, g, be, w2, b2, ew_out):
    blk = ew_out.shape[0]
    t = _mish(_ln(jnp.dot(rel_ref[:], w1[:], preferred_element_type=jnp.float32)
                  + b1[:], g[:], be[:]))
    w = jnp.dot(t, w2[:], preferred_element_type=jnp.float32) + b2[:]
    row = (pl.program_id(0) * blk
           + jax.lax.broadcasted_iota(jnp.int32, (blk, 1), 0))
    ew_out[:] = jnp.where(row < E, jnp.exp(w), 0.0)


def _edge_call(rel, p):
    blk = 512
    grid = (pl.cdiv(EPAD, blk),)
    row_spec = pl.BlockSpec((blk, D), lambda i: (i, 0))
    w_spec = pl.BlockSpec((D, D), lambda i: (0, 0))
    v_spec = pl.BlockSpec((1, D), lambda i: (0, 0))
    return pl.pallas_call(
        _edge_body,
        grid=grid,
        in_specs=[row_spec, w_spec, v_spec, v_spec, v_spec, w_spec, v_spec],
        out_specs=row_spec,
        out_shape=jax.ShapeDtypeStruct((EPAD, D), jnp.float32),
    )(rel, p['we_W1'], p['we_b1'].reshape(1, D), p['we_g'].reshape(1, D),
      p['we_be'].reshape(1, D), p['we_W2'], p['we_b2'].reshape(1, D))


def _scatter_body(ew_hbm, dst_hbm, s_out_hbm,
                  didx_v, eid_v, nloc_v, eid_g, rows_v, acc_v, sem):
    c = lax.axis_index("c")
    s = lax.axis_index("s")
    wid = s * NC + c
    lo = wid * OWN

    def zbody(i, carry):
        for j in range(D // L):
            acc_v[i, pl.ds(j * L, L)] = jnp.zeros((L,), jnp.float32)
        return carry

    lax.fori_loop(0, OWN, zbody, 0)

    iota = lax.iota(jnp.int32, L)

    def chunk_body(ci, carry):
        base = ((ci + wid * 2) % NCHUNK_S) * CSC
        pltpu.sync_copy(dst_hbm.at[pl.ds(base, CSC)], didx_v)

        def scan_body(i, m):
            dv = didx_v[pl.ds(i * L, L)]
            lv = dv - lo
            ok = (lv >= 0) & (lv < OWN)
            plsc.store_compressed(eid_v.at[pl.ds(m, L)], base + i * L + iota,
                                  mask=ok)
            plsc.store_compressed(nloc_v.at[pl.ds(m, L)], lv, mask=ok)
            pc = plsc.all_reduce_population_count(ok)
            return m + pc[0]

        m = lax.fori_loop(0, CSC // L, scan_body, jnp.int32(0))

        for t in range(GRP // L):
            eid_v[pl.ds(m + t * L, L)] = jnp.full((L,), E, jnp.int32)
            nloc_v[pl.ds(m + t * L, L)] = jnp.zeros((L,), jnp.int32)

        def group_work(g):
            for r in range(GRP // L):
                eid_g[pl.ds(r * L, L)] = eid_v[pl.ds(g * GRP + r * L, L)]
            pltpu.async_copy(ew_hbm.at[eid_g], rows_v, sem).wait()

            def blk16(k, carry3):
                nv = nloc_v[pl.ds(g * GRP + k * L, L)]
                for r in range(L):
                    n = nv[r]
                    row = k * L + r
                    for j in range(D // L):
                        plsc.addupdate(acc_v.at[n, pl.ds(j * L, L)],
                                       rows_v[row, pl.ds(j * L, L)])
                return carry3

            lax.fori_loop(0, GRP // L, blk16, 0)

        ng = (m + GRP - 1) // GRP
        for gs in range(2):
            @pl.when(gs < ng)
            def _():
                group_work(gs)

        def group_body(g, carry2):
            group_work(g)
            return carry2

        lax.fori_loop(2, ng, group_body, 0)
        return carry

    lax.fori_loop(0, NCHUNK_S, chunk_body, 0)
    pltpu.sync_copy(acc_v, s_out_hbm.at[pl.ds(lo, OWN)])


def _scatter_call(ew, dst):
    mesh = plsc.VectorSubcoreMesh(core_axis_name="c", subcore_axis_name="s",
                                  num_cores=NC, num_subcores=NS)
    out = pl.kernel(
        _scatter_body,
        out_type=jax.ShapeDtypeStruct((NPAD, D), jnp.float32),
        mesh=mesh,
        compiler_params=pltpu.CompilerParams(needs_layout_passes=False),
        scratch_types=[
            pltpu.VMEM((CSC,), jnp.int32),
            pltpu.VMEM((CSC + 2 * GRP,), jnp.int32),
            pltpu.VMEM((CSC + 2 * GRP,), jnp.int32),
            pltpu.VMEM((GRP,), jnp.int32),
            pltpu.VMEM((GRP, D), jnp.float32),
            pltpu.VMEM((OWN, D), jnp.float32),
            pltpu.SemaphoreType.DMA,
        ],
    )(ew, dst)
    return out[:N]


def _colsum_body(s_ref, out_ref):
    out_ref[:] = jnp.sum(s_ref[:], axis=0, keepdims=True)


def _colsum_call(S):
    return pl.pallas_call(
        _colsum_body,
        out_shape=jax.ShapeDtypeStruct((1, D), jnp.float32),
    )(S)


def _final_body(nf_ref, s_ref, cs_ref, w1a, w1b, b1, w2, b2, h_out):
    hn = s_ref[:] / cs_ref[:]
    u = (jnp.dot(nf_ref[:], w1a[:], preferred_element_type=jnp.float32)
         + jnp.dot(hn, w1b[:], preferred_element_type=jnp.float32) + b1[:])
    h_out[:] = jnp.dot(_mish(u), w2[:], preferred_element_type=jnp.float32) + b2[:]


def _final_call(node_feat, S, colsum, p):
    blk = 512
    grid = (pl.cdiv(N, blk),)
    row_spec = pl.BlockSpec((blk, D), lambda i: (i, 0))
    w_spec = pl.BlockSpec((D, D), lambda i: (0, 0))
    v_spec = pl.BlockSpec((1, D), lambda i: (0, 0))
    return pl.pallas_call(
        _final_body,
        grid=grid,
        in_specs=[row_spec, row_spec, v_spec,
                  w_spec, w_spec, v_spec, w_spec, v_spec],
        out_specs=row_spec,
        out_shape=jax.ShapeDtypeStruct((N, D), jnp.float32),
    )(node_feat, S, colsum,
      p['nm_W1'][:D], p['nm_W1'][D:], p['nm_b1'].reshape(1, D),
      p['nm_W2'], p['nm_b2'].reshape(1, D))


@jax.jit
def _run(node_feat, coord_feat, params, edge_index):
    src = edge_index[0]
    dst = edge_index[1]
    q, kpb, x = _node_call(coord_feat, params)
    rel = _gather_call(q, kpb, src, dst)
    ew = _edge_call(rel, params)
    S = _scatter_call(ew, dst)
    colsum = _colsum_call(S)
    h = _final_call(node_feat, S, colsum, params)
    return (h, x)


def kernel(node_feat, coord_feat, params, edge_index):
    return _run(node_feat, coord_feat, params, edge_index)


# speedup vs baseline: 5.4167x; 2.8422x over previous
import functools

import jax
import jax.numpy as jnp
from jax import lax
from jax.experimental import pallas as pl
from jax.experimental.pallas import tpu as pltpu
from jax.experimental.pallas import tpu_sc as plsc

N = 10000
E = 160000
D = 256

NC = 2
NS = 16
L = 16
NW = NC * NS

EPW = E // NW
CG = 40
NCHUNK_G = EPW // CG

OWN = 320
NPAD = NW * OWN
CSC = 2000
NCHUNK_S = E // CSC
GRP = 64
EPAD = E + GRP


def _mish(x):
    return x * jnp.tanh(jax.nn.softplus(x))


def _ln(x, g, b):
    mu = jnp.mean(x, axis=-1, keepdims=True)
    var = jnp.var(x, axis=-1, keepdims=True)
    return (x - mu) / jnp.sqrt(var + 1e-5) * g + b


def _node_body(c_ref, wq, bq, gq, nq, wk, bk, gk, nk,
               pw1, pb1, pg, pe, pw2, pb2,
               bw1, bb1, bg, be, bw2, bb2,
               cw1, cb1, cw2,
               q_out, kpb_out, x_out):
    c = c_ref[:]
    q = _mish(_ln(jnp.dot(c, wq[:], preferred_element_type=jnp.float32) + bq[:],
                  gq[:], nq[:]))
    q_out[:] = q
    k = _mish(_ln(jnp.dot(c, wk[:], preferred_element_type=jnp.float32) + bk[:],
                  gk[:], nk[:]))
    pem = jnp.dot(
        _mish(_ln(jnp.dot(c, pw1[:], preferred_element_type=jnp.float32) + pb1[:],
                  pg[:], pe[:])),
        pw2[:], preferred_element_type=jnp.float32) + pb2[:]
    peb = jnp.dot(
        _mish(_ln(jnp.dot(c, bw1[:], preferred_element_type=jnp.float32) + bb1[:],
                  bg[:], be[:])),
        bw2[:], preferred_element_type=jnp.float32) + bb2[:]
    kpb_out[:, 0:D] = k
    kpb_out[:, D:2 * D] = pem
    kpb_out[:, 2 * D:3 * D] = peb
    x_out[:] = jnp.dot(_mish(jnp.dot(c, cw1[:], preferred_element_type=jnp.float32)
                             + cb1[:]),
                       cw2[:], preferred_element_type=jnp.float32)


def _node_call(coord, p):
    blk = 512
    grid = (pl.cdiv(N, blk),)
    row_spec = pl.BlockSpec((blk, D), lambda i: (i, 0))
    w_spec = pl.BlockSpec((D, D), lambda i: (0, 0))
    v_spec = pl.BlockSpec((1, D), lambda i: (0, 0))
    vecs = lambda *names: [p[n].reshape(1, D) for n in names]
    return pl.pallas_call(
        _node_body,
        grid=grid,
        in_specs=[row_spec,
                  w_spec, v_spec, v_spec, v_spec,
                  w_spec, v_spec, v_spec, v_spec,
                  w_spec, v_spec, v_spec, v_spec, w_spec, v_spec,
                  w_spec, v_spec, v_spec, v_spec, w_spec, v_spec,
                  w_spec, v_spec, w_spec],
        out_specs=[row_spec,
                   pl.BlockSpec((blk, 3 * D), lambda i: (i, 0)),
                   row_spec],
        out_shape=[jax.ShapeDtypeStruct((N, D), jnp.float32),
                   jax.ShapeDtypeStruct((N, 3 * D), jnp.float32),
                   jax.ShapeDtypeStruct((N, D), jnp.float32)],
    )(coord,
      p['Wq'], *vecs('bq', 'gq', 'betaq'),
      p['Wk'], *vecs('bk', 'gk', 'betak'),
      p['pm_W1'], *vecs('pm_b1', 'pm_g', 'pm_be'), p['pm_W2'], p['pm_b2'].reshape(1, D),
      p['pb_W1'], *vecs('pb_b1', 'pb_g', 'pb_be'), p['pb_W2'], p['pb_b2'].reshape(1, D),
      p['cm_W1'], p['cm_b1'].reshape(1, D), p['cm_W2'])


def _edge_rel_compute(qrows_v, kpbrows_v, out_v, b):
    @functools.partial(plsc.parallel_loop, b * CG, (b + 1) * CG, unroll=2)
    def edge_body(e):
        dvs = []
        acc = jnp.zeros((L,), jnp.float32)
        for j in range(D // L):
            qv = qrows_v[e, pl.ds(j * L, L)]
            kv = kpbrows_v[e, pl.ds(j * L, L)]
            dv = qv - kv
            dvs.append(dv)
            acc = acc + dv * dv
        iota = lax.iota(jnp.int32, L)
        for sh in (8, 4, 2, 1):
            acc = acc + acc.at[iota ^ sh].get(mode='promise_in_bounds',
                                              unique_indices=True)
        x = acc + 1e-8
        yi = jnp.int32(0x5F3759DF) - (plsc.bitcast(x, jnp.int32) >> 1)
        y = plsc.bitcast(yi, jnp.float32)
        for _ in range(3):
            y = y * (1.5 - 0.5 * x * y * y)
        for j in range(D // L):
            pv = kpbrows_v[e, pl.ds(D + j * L, L)]
            bv = kpbrows_v[e, pl.ds(2 * D + j * L, L)]
            out_v[e, pl.ds(j * L, L)] = dvs[j] * y * pv + bv


def _gather_body(q_hbm, kpb_hbm, src_hbm, dst_hbm, rel_hbm,
                 sidx_v, didx_v, qrows_v, kpbrows_v, out_v,
                 semg0, semg1, semw0, semw1):
    c = lax.axis_index("c")
    s = lax.axis_index("s")
    wid = s * NC + c
    base_w = wid * EPW
    semg = (semg0, semg1)
    semw = (semw0, semw1)

    pltpu.sync_copy(src_hbm.at[pl.ds(base_w, EPW)], sidx_v)
    pltpu.sync_copy(dst_hbm.at[pl.ds(base_w, EPW)], didx_v)

    def start_gather(ci, b):
        pltpu.async_copy(q_hbm.at[sidx_v.at[pl.ds(ci * CG, CG)]],
                         qrows_v.at[pl.ds(b * CG, CG)], semg[b])
        pltpu.async_copy(kpb_hbm.at[didx_v.at[pl.ds(ci * CG, CG)]],
                         kpbrows_v.at[pl.ds(b * CG, CG)], semg[b])

    def wait_gather(ci, b):
        pltpu.make_async_copy(q_hbm.at[sidx_v.at[pl.ds(ci * CG, CG)]],
                              qrows_v.at[pl.ds(b * CG, CG)], semg[b]).wait()
        pltpu.make_async_copy(kpb_hbm.at[didx_v.at[pl.ds(ci * CG, CG)]],
                              kpbrows_v.at[pl.ds(b * CG, CG)], semg[b]).wait()

    def start_write(ci, b):
        pltpu.async_copy(out_v.at[pl.ds(b * CG, CG)],
                         rel_hbm.at[pl.ds(base_w + ci * CG, CG)], semw[b])

    def wait_write(ci, b):
        pltpu.make_async_copy(out_v.at[pl.ds(b * CG, CG)],
                              rel_hbm.at[pl.ds(base_w + ci * CG, CG)],
                              semw[b]).wait()

    start_gather(0, 0)
    start_gather(1, 1)
    start_write(0, 0)
    start_write(1, 1)

    def outer(cj, carry):
        for b in range(2):
            ci = cj * 2 + b
            wait_gather(ci, b)
            wait_write(ci, b)
            _edge_rel_compute(qrows_v, kpbrows_v, out_v, b)
            start_write(ci, b)

            @pl.when(ci + 2 < NCHUNK_G)
            def _():
                start_gather(ci + 2, b)

        return carry

    lax.fori_loop(0, NCHUNK_G // 2, outer, 0)

    ci = NCHUNK_G - 1
    wait_gather(ci, 0)
    wait_write(ci, 0)
    _edge_rel_compute(qrows_v, kpbrows_v, out_v, 0)
    start_write(ci, 0)
    wait_write(ci, 0)
    wait_write(ci, 1)


def _gather_call(q, kpb, src, dst):
    mesh = plsc.VectorSubcoreMesh(core_axis_name="c", subcore_axis_name="s",
                                  num_cores=NC, num_subcores=NS)
    return pl.kernel(
        _gather_body,
        out_type=jax.ShapeDtypeStruct((E, D), jnp.float32),
        mesh=mesh,
        compiler_params=pltpu.CompilerParams(needs_layout_passes=False),
        scratch_types=[
            pltpu.VMEM((EPW,), jnp.int32),
            pltpu.VMEM((EPW,), jnp.int32),
            pltpu.VMEM((2 * CG, D), jnp.float32),
            pltpu.VMEM((2 * CG, 3 * D), jnp.float32),
            pltpu.VMEM((2 * CG, D), jnp.float32),
            pltpu.SemaphoreType.DMA,
            pltpu.SemaphoreType.DMA,
            pltpu.SemaphoreType.DMA,
            pltpu.SemaphoreType.DMA,
        ],
    )(q, kpb, src, dst)


def _edge_body(rel_ref, w1, b1, g, be, w2, b2, ew_out):
    blk = ew_out.shape[0]
    t = _mish(_ln(jnp.dot(rel_ref[:], w1[:], preferred_element_type=jnp.float32)
                  + b1[:], g[:], be[:]))
    w = jnp.dot(t, w2[:], preferred_element_type=jnp.float32) + b2[:]
    row = (pl.program_id(0) * blk
           + jax.lax.broadcasted_iota(jnp.int32, (blk, 1), 0))
    ew_out[:] = jnp.where(row < E, jnp.exp(w), 0.0)


def _edge_call(rel, p):
    blk = 512
    grid = (pl.cdiv(EPAD, blk),)
    row_spec = pl.BlockSpec((blk, D), lambda i: (i, 0))
    w_spec = pl.BlockSpec((D, D), lambda i: (0, 0))
    v_spec = pl.BlockSpec((1, D), lambda i: (0, 0))
    return pl.pallas_call(
        _edge_body,
        grid=grid,
        in_specs=[row_spec, w_spec, v_spec, v_spec, v_spec, w_spec, v_spec],
        out_specs=row_spec,
        out_shape=jax.ShapeDtypeStruct((EPAD, D), jnp.float32),
    )(rel, p['we_W1'], p['we_b1'].reshape(1, D), p['we_g'].reshape(1, D),
      p['we_be'].reshape(1, D), p['we_W2'], p['we_b2'].reshape(1, D))


def _scatter_body(ew_hbm, dst_hbm, s_out_hbm,
                  didx_v, eid_v, nloc_v, eid_g, rows_v, acc_v, sem):
    c = lax.axis_index("c")
    s = lax.axis_index("s")
    wid = s * NC + c
    lo = wid * OWN

    def zbody(i, carry):
        for j in range(D // L):
            acc_v[i, pl.ds(j * L, L)] = jnp.zeros((L,), jnp.float32)
        return carry

    lax.fori_loop(0, OWN, zbody, 0)

    iota = lax.iota(jnp.int32, L)

    def group_work(g):
        for r in range(GRP // L):
            eid_g[pl.ds(r * L, L)] = eid_v[pl.ds(g * GRP + r * L, L)]
        pltpu.async_copy(ew_hbm.at[eid_g], rows_v, sem).wait()

        def blk16(k, carry3):
            nv = nloc_v[pl.ds(g * GRP + k * L, L)]
            for r in range(L):
                n = nv[r]
                row = k * L + r
                for j in range(D // L):
                    plsc.addupdate(acc_v.at[n, pl.ds(j * L, L)],
                                   rows_v[row, pl.ds(j * L, L)])
            return carry3

        lax.fori_loop(0, GRP // L, blk16, 0)

    def chunk_body(ci, m_in):
        base = ((ci + wid * 2) % NCHUNK_S) * CSC
        pltpu.sync_copy(dst_hbm.at[pl.ds(base, CSC)], didx_v)

        def scan_body(i, m):
            dv = didx_v[pl.ds(i * L, L)]
            lv = dv - lo
            ok = (lv >= 0) & (lv < OWN)
            plsc.store_compressed(eid_v.at[pl.ds(m, L)], base + i * L + iota,
                                  mask=ok)
            plsc.store_compressed(nloc_v.at[pl.ds(m, L)], lv, mask=ok)
            pc = plsc.all_reduce_population_count(ok)
            return m + pc[0]

        m2 = lax.fori_loop(0, CSC // L, scan_body, m_in)

        nfull = m2 // GRP

        def group_body(g, carry2):
            group_work(g)
            return carry2

        lax.fori_loop(0, nfull, group_body, 0)

        for r in range(GRP // L):
            ev = eid_v[pl.ds(nfull * GRP + r * L, L)]
            nv = nloc_v[pl.ds(nfull * GRP + r * L, L)]
            eid_v[pl.ds(r * L, L)] = ev
            nloc_v[pl.ds(r * L, L)] = nv

        return m2 - nfull * GRP

    m_fin = lax.fori_loop(0, NCHUNK_S, chunk_body, jnp.int32(0))

    for t in range(GRP // L):
        eid_v[pl.ds(m_fin + t * L, L)] = jnp.full((L,), E, jnp.int32)
        nloc_v[pl.ds(m_fin + t * L, L)] = jnp.zeros((L,), jnp.int32)

    @pl.when(m_fin > 0)
    def _():
        group_work(0)

    pltpu.sync_copy(acc_v, s_out_hbm.at[pl.ds(lo, OWN)])


def _scatter_call(ew, dst):
    mesh = plsc.VectorSubcoreMesh(core_axis_name="c", subcore_axis_name="s",
                                  num_cores=NC, num_subcores=NS)
    out = pl.kernel(
        _scatter_body,
        out_type=jax.ShapeDtypeStruct((NPAD, D), jnp.float32),
        mesh=mesh,
        compiler_params=pltpu.CompilerParams(needs_layout_passes=False),
        scratch_types=[
            pltpu.VMEM((CSC,), jnp.int32),
            pltpu.VMEM((CSC + 2 * GRP,), jnp.int32),
            pltpu.VMEM((CSC + 2 * GRP,), jnp.int32),
            pltpu.VMEM((GRP,), jnp.int32),
            pltpu.VMEM((GRP, D), jnp.float32),
            pltpu.VMEM((OWN, D), jnp.float32),
            pltpu.SemaphoreType.DMA,
        ],
    )(ew, dst)
    return out[:N]


def _colsum_body(s_ref, out_ref):
    out_ref[:] = jnp.sum(s_ref[:], axis=0, keepdims=True)


def _colsum_call(S):
    return pl.pallas_call(
        _colsum_body,
        out_shape=jax.ShapeDtypeStruct((1, D), jnp.float32),
    )(S)


def _final_body(nf_ref, s_ref, cs_ref, w1a, w1b, b1, w2, b2, h_out):
    hn = s_ref[:] / cs_ref[:]
    u = (jnp.dot(nf_ref[:], w1a[:], preferred_element_type=jnp.float32)
         + jnp.dot(hn, w1b[:], preferred_element_type=jnp.float32) + b1[:])
    h_out[:] = jnp.dot(_mish(u), w2[:], preferred_element_type=jnp.float32) + b2[:]


def _final_call(node_feat, S, colsum, p):
    blk = 512
    grid = (pl.cdiv(N, blk),)
    row_spec = pl.BlockSpec((blk, D), lambda i: (i, 0))
    w_spec = pl.BlockSpec((D, D), lambda i: (0, 0))
    v_spec = pl.BlockSpec((1, D), lambda i: (0, 0))
    return pl.pallas_call(
        _final_body,
        grid=grid,
        in_specs=[row_spec, row_spec, v_spec,
                  w_spec, w_spec, v_spec, w_spec, v_spec],
        out_specs=row_spec,
        out_shape=jax.ShapeDtypeStruct((N, D), jnp.float32),
    )(node_feat, S, colsum,
      p['nm_W1'][:D], p['nm_W1'][D:], p['nm_b1'].reshape(1, D),
      p['nm_W2'], p['nm_b2'].reshape(1, D))


@jax.jit
def _run(node_feat, coord_feat, params, edge_index):
    src = edge_index[0]
    dst = edge_index[1]
    q, kpb, x = _node_call(coord_feat, params)
    rel = _gather_call(q, kpb, src, dst)
    ew = _edge_call(rel, params)
    S = _scatter_call(ew, dst)
    colsum = _colsum_call(S)
    h = _final_call(node_feat, S, colsum, params)
    return (h, x)


def kernel(node_feat, coord_feat, params, edge_index):
    return _run(node_feat, coord_feat, params, edge_index)
